# baseline (speedup 1.0000x reference)
import jax
import jax.numpy as jnp
from jax import lax
from jax.experimental import pallas as pl
from jax.experimental.pallas import tpu as pltpu
from jax.experimental.pallas import tpu_sc as plsc

NUM_ROWS = 320000
D = 128
NUM_SEG = 10000
NUM_CORES = 2
NUM_SUBCORES = 16
CHUNK = 128
WINDOWS_PER_CORE = NUM_ROWS // (NUM_CORES * CHUNK)
ACC_ROWS = 10112
ZROWS = ACC_ROWS // NUM_SUBCORES
NRING = 2
HDR = 8
STAGE_ROWS = 32
DUMP_BASE = ACC_ROWS - 8


def _phase1_body(
    x_hbm, batch_hbm, zeros_hbm, a0_hbm, a1_hbm,
    xbuf0, xbuf1, ibuf0, ibuf1, stage, r16, r32, rraw, acc,
    semx0, semx1, semi0, semi1,
):
    c = lax.axis_index("c")
    s = lax.axis_index("s")
    xbufs = (xbuf0, xbuf1)
    ibufs = (ibuf0, ibuf1)
    semxs = (semx0, semx1)
    semis = (semi0, semi1)
    iota = lax.iota(jnp.int32, 16)

    pltpu.sync_copy(zeros_hbm, xbuf0)
    for z in range(ZROWS // CHUNK):
        pltpu.sync_copy(xbuf0, acc.at[pl.ds(ZROWS * s + z * CHUNK, CHUNK)])
    rem = ZROWS % CHUNK
    if rem:
        pltpu.sync_copy(
            xbuf0.at[pl.ds(0, rem)],
            acc.at[pl.ds(ZROWS * s + ZROWS - rem, rem)],
        )
    plsc.subcore_barrier()

    num_windows = (WINDOWS_PER_CORE - s + NUM_SUBCORES - 1) // NUM_SUBCORES
    row0 = c * (WINDOWS_PER_CORE * CHUNK)

    def window_q(t):
        return pl.multiple_of(row0 + (s + t * NUM_SUBCORES) * CHUNK, 8)

    def issue_load(t, b):
        q = window_q(t)
        pltpu.async_copy(x_hbm.at[pl.ds(q, CHUNK)], xbufs[b], semxs[b])
        pltpu.async_copy(
            batch_hbm.at[pl.ds(q, CHUNK)], ibufs[b].at[pl.ds(HDR, CHUNK)], semis[b]
        )

    ibuf0[pl.ds(0, 16)] = jnp.full((16,), -1, jnp.int32)
    ibuf1[pl.ds(0, 16)] = jnp.full((16,), -1, jnp.int32)

    @pl.when(num_windows > 0)
    def _():
        issue_load(0, 0)

    def process(t, b):
        @pl.when(t + 1 < num_windows)
        def _():
            issue_load(t + 1, 1 - b)

        q = window_q(t)
        pltpu.make_async_copy(
            batch_hbm.at[pl.ds(q, CHUNK)], ibufs[b].at[pl.ds(HDR, CHUNK)], semis[b]
        ).wait()
        pltpu.make_async_copy(
            x_hbm.at[pl.ds(q, CHUNK)], xbufs[b], semxs[b]
        ).wait()

        dumpv = DUMP_BASE + jnp.bitwise_and(iota, 7)
        for j in range(CHUNK // 16):
            rraw[pl.ds(j * 16, 16)] = dumpv

        zero16 = jnp.zeros((16,), jnp.float32)

        def group(g, carry):
            n = carry[0]
            accv = list(carry[1:])
            curv = ibufs[b][pl.ds(HDR + g * 16, 16)]
            prvv = ibufs[b][pl.ds(HDR - 1 + g * 16, 16)]
            mv = curv != prvv
            mi = jnp.where(mv, 1, 0)
            posv = jnp.zeros((16,), jnp.int32)
            for l in range(16):
                r = g * 16 + l
                bnd = mi[l]
                n = n + bnd
                posv = jnp.where(iota == l, n - 1, posv)
                rid = jnp.minimum(n - 1, STAGE_ROWS - 1)
                keep = (1 - bnd).astype(jnp.float32)
                for j in range(D // 16):
                    v = xbufs[b][r, pl.ds(j * 16, 16)]
                    accv[j] = v + accv[j] * keep
                    stage[rid, pl.ds(j * 16, 16)] = accv[j]
            plsc.store_scatter(
                rraw, [jnp.minimum(posv, CHUNK - 1)], curv, mask=mv
            )
            return tuple([n] + accv)

        init = tuple([jnp.int32(0)] + [zero16] * (D // 16))
        nruns = lax.fori_loop(0, CHUNK // 16, group, init)[0]

        @pl.when(nruns <= 16)
        def _():
            r16[pl.ds(0, 16)] = rraw[pl.ds(0, 16)]
            pltpu.sync_copy(stage.at[pl.ds(0, 16)], acc.at[r16], add=True)

        @pl.when((nruns > 16) & (nruns <= STAGE_ROWS))
        def _():
            r32[pl.ds(0, 16)] = rraw[pl.ds(0, 16)]
            r32[pl.ds(16, 16)] = rraw[pl.ds(16, 16)]
            pltpu.sync_copy(stage.at[pl.ds(0, 32)], acc.at[r32], add=True)

        @pl.when(nruns > STAGE_ROWS)
        def _():
            for j in range(CHUNK // 16):
                rraw[pl.ds(j * 16, 16)] = ibufs[b][pl.ds(HDR + j * 16, 16)]
            pltpu.sync_copy(xbufs[b], acc.at[rraw], add=True)

    def duo(g, carry):
        for b in range(NRING):
            t = NRING * g + b

            @pl.when(t < num_windows)
            def _():
                process(t, b)

        return carry

    lax.fori_loop(0, (num_windows + NRING - 1) // NRING, duo, 0)
    plsc.subcore_barrier()

    @pl.when(c == 0)
    def _():
        pltpu.sync_copy(
            acc.at[pl.ds(ZROWS * s, ZROWS)], a0_hbm.at[pl.ds(ZROWS * s, ZROWS)]
        )

    @pl.when(c == 1)
    def _():
        pltpu.sync_copy(
            acc.at[pl.ds(ZROWS * s, ZROWS)], a1_hbm.at[pl.ds(ZROWS * s, ZROWS)]
        )


def _add_body(a_ref, b_ref, o_ref):
    o_ref[...] = a_ref[...] + b_ref[...]


@jax.jit
def kernel(x, batch):
    batch = batch.astype(jnp.int32)
    zeros = jnp.zeros((CHUNK, D), jnp.float32)

    mesh = plsc.VectorSubcoreMesh(core_axis_name="c", subcore_axis_name="s")
    phase1 = pl.kernel(
        _phase1_body,
        mesh=mesh,
        compiler_params=pltpu.CompilerParams(needs_layout_passes=False),
        out_type=(
            jax.ShapeDtypeStruct((ACC_ROWS, D), jnp.float32),
            jax.ShapeDtypeStruct((ACC_ROWS, D), jnp.float32),
        ),
        scratch_types=[
            pltpu.VMEM((CHUNK, D), jnp.float32),
            pltpu.VMEM((CHUNK, D), jnp.float32),
            pltpu.VMEM((CHUNK + HDR,), jnp.int32),
            pltpu.VMEM((CHUNK + HDR,), jnp.int32),
            pltpu.VMEM((STAGE_ROWS, D), jnp.float32),
            pltpu.VMEM((16,), jnp.int32),
            pltpu.VMEM((32,), jnp.int32),
            pltpu.VMEM((CHUNK,), jnp.int32),
            pltpu.VMEM_SHARED((ACC_ROWS, D), jnp.float32),
            pltpu.SemaphoreType.DMA,
            pltpu.SemaphoreType.DMA,
            pltpu.SemaphoreType.DMA,
            pltpu.SemaphoreType.DMA,
        ],
    )
    a0, a1 = phase1(x, batch, zeros)

    blk = 1000
    out = pl.pallas_call(
        _add_body,
        grid=(NUM_SEG // blk,),
        in_specs=[
            pl.BlockSpec((blk, D), lambda i: (i, 0)),
            pl.BlockSpec((blk, D), lambda i: (i, 0)),
        ],
        out_specs=pl.BlockSpec((blk, D), lambda i: (i, 0)),
        out_shape=jax.ShapeDtypeStruct((NUM_SEG, D), jnp.float32),
    )(a0, a1)
    return out


# speedup vs baseline: 3.3733x; 3.3733x over previous
import jax
import jax.numpy as jnp
from jax import lax
from jax.experimental import pallas as pl
from jax.experimental.pallas import tpu as pltpu
from jax.experimental.pallas import tpu_sc as plsc

NUM_ROWS = 320000
D = 128
NUM_SEG = 10000
NUM_CORES = 2
NUM_SUBCORES = 16
CHUNK = 128
WINDOWS_PER_CORE = NUM_ROWS // (NUM_CORES * CHUNK)
ACC_ROWS = 10112
ZROWS = ACC_ROWS // NUM_SUBCORES
NRING = 3


def _phase1_body(
    x_hbm, batch_hbm, zeros_hbm, a0_hbm, a1_hbm,
    xbuf0, xbuf1, xbuf2, ibuf0, ibuf1, ibuf2, acc,
    semx0, semx1, semx2, semi0, semi1, semi2,
):
    c = lax.axis_index("c")
    s = lax.axis_index("s")
    xbufs = (xbuf0, xbuf1, xbuf2)
    ibufs = (ibuf0, ibuf1, ibuf2)
    semxs = (semx0, semx1, semx2)
    semis = (semi0, semi1, semi2)

    pltpu.sync_copy(zeros_hbm, xbuf0)
    for z in range(ZROWS // CHUNK):
        pltpu.sync_copy(xbuf0, acc.at[pl.ds(ZROWS * s + z * CHUNK, CHUNK)])
    rem = ZROWS % CHUNK
    if rem:
        pltpu.sync_copy(
            xbuf0.at[pl.ds(0, rem)],
            acc.at[pl.ds(ZROWS * s + ZROWS - rem, rem)],
        )
    plsc.subcore_barrier()

    num_windows = (WINDOWS_PER_CORE - s + NUM_SUBCORES - 1) // NUM_SUBCORES
    row0 = c * (WINDOWS_PER_CORE * CHUNK)

    def window_q(t):
        return pl.multiple_of(row0 + (s + t * NUM_SUBCORES) * CHUNK, 8)

    def issue_load(t, b):
        q = window_q(t)
        pltpu.async_copy(x_hbm.at[pl.ds(q, CHUNK)], xbufs[b], semxs[b])
        pltpu.async_copy(batch_hbm.at[pl.ds(q, CHUNK)], ibufs[b], semis[b])

    for b in range(NRING - 1):
        @pl.when(b < num_windows)
        def _():
            issue_load(b, b)

    def tri(g, carry):
        for b in range(NRING):
            t = NRING * g + b

            @pl.when(t < num_windows)
            def _():
                @pl.when(t + NRING - 1 < num_windows)
                def _():
                    issue_load(t + NRING - 1, (b + NRING - 1) % NRING)

                q = window_q(t)
                pltpu.make_async_copy(
                    batch_hbm.at[pl.ds(q, CHUNK)], ibufs[b], semis[b]
                ).wait()
                pltpu.make_async_copy(
                    x_hbm.at[pl.ds(q, CHUNK)], xbufs[b], semxs[b]
                ).wait()
                pltpu.sync_copy(xbufs[b], acc.at[ibufs[b]], add=True)

        return carry

    lax.fori_loop(0, (num_windows + NRING - 1) // NRING, tri, 0)
    plsc.subcore_barrier()

    @pl.when(c == 0)
    def _():
        pltpu.sync_copy(
            acc.at[pl.ds(ZROWS * s, ZROWS)], a0_hbm.at[pl.ds(ZROWS * s, ZROWS)]
        )

    @pl.when(c == 1)
    def _():
        pltpu.sync_copy(
            acc.at[pl.ds(ZROWS * s, ZROWS)], a1_hbm.at[pl.ds(ZROWS * s, ZROWS)]
        )


def _add_body(a_ref, b_ref, o_ref):
    o_ref[...] = a_ref[...] + b_ref[...]


@jax.jit
def kernel(x, batch):
    batch = batch.astype(jnp.int32)
    zeros = jnp.zeros((CHUNK, D), jnp.float32)

    mesh = plsc.VectorSubcoreMesh(core_axis_name="c", subcore_axis_name="s")
    phase1 = pl.kernel(
        _phase1_body,
        mesh=mesh,
        out_type=(
            jax.ShapeDtypeStruct((ACC_ROWS, D), jnp.float32),
            jax.ShapeDtypeStruct((ACC_ROWS, D), jnp.float32),
        ),
        scratch_types=[
            pltpu.VMEM((CHUNK, D), jnp.float32),
            pltpu.VMEM((CHUNK, D), jnp.float32),
            pltpu.VMEM((CHUNK, D), jnp.float32),
            pltpu.VMEM((CHUNK,), jnp.int32),
            pltpu.VMEM((CHUNK,), jnp.int32),
            pltpu.VMEM((CHUNK,), jnp.int32),
            pltpu.VMEM_SHARED((ACC_ROWS, D), jnp.float32),
            pltpu.SemaphoreType.DMA,
            pltpu.SemaphoreType.DMA,
            pltpu.SemaphoreType.DMA,
            pltpu.SemaphoreType.DMA,
            pltpu.SemaphoreType.DMA,
            pltpu.SemaphoreType.DMA,
        ],
    )
    a0, a1 = phase1(x, batch, zeros)

    blk = 2000
    out = pl.pallas_call(
        _add_body,
        grid=(NUM_SEG // blk,),
        in_specs=[
            pl.BlockSpec((blk, D), lambda i: (i, 0)),
            pl.BlockSpec((blk, D), lambda i: (i, 0)),
        ],
        out_specs=pl.BlockSpec((blk, D), lambda i: (i, 0)),
        out_shape=jax.ShapeDtypeStruct((NUM_SEG, D), jnp.float32),
    )(a0, a1)
    return out


# speedup vs baseline: 3.3757x; 1.0007x over previous
import jax
import jax.numpy as jnp
from jax import lax
from jax.experimental import pallas as pl
from jax.experimental.pallas import tpu as pltpu
from jax.experimental.pallas import tpu_sc as plsc

NUM_ROWS = 320000
D = 128
NUM_SEG = 10000
NUM_CORES = 2
NUM_SUBCORES = 16
CHUNK = 128
WINDOWS_PER_CORE = NUM_ROWS // (NUM_CORES * CHUNK)
ACC_ROWS = 10112
ZROWS = ACC_ROWS // NUM_SUBCORES
NRING = 3


def _phase1_body(
    x_hbm, batch_hbm, zeros_hbm, a0_hbm, a1_hbm,
    xbuf0, xbuf1, xbuf2, ibuf0, ibuf1, ibuf2, acc,
    semx0, semx1, semx2, semi0, semi1, semi2, sems0, sems1, sems2,
):
    c = lax.axis_index("c")
    s = lax.axis_index("s")
    xbufs = (xbuf0, xbuf1, xbuf2)
    ibufs = (ibuf0, ibuf1, ibuf2)
    semxs = (semx0, semx1, semx2)
    semis = (semi0, semi1, semi2)
    semss = (sems0, sems1, sems2)

    pltpu.sync_copy(zeros_hbm, xbuf0)
    for z in range(ZROWS // CHUNK):
        pltpu.sync_copy(xbuf0, acc.at[pl.ds(ZROWS * s + z * CHUNK, CHUNK)])
    rem = ZROWS % CHUNK
    if rem:
        pltpu.sync_copy(
            xbuf0.at[pl.ds(0, rem)],
            acc.at[pl.ds(ZROWS * s + ZROWS - rem, rem)],
        )
    plsc.subcore_barrier()

    num_windows = (WINDOWS_PER_CORE - s + NUM_SUBCORES - 1) // NUM_SUBCORES
    row0 = c * (WINDOWS_PER_CORE * CHUNK)

    def window_q(t):
        return pl.multiple_of(row0 + (s + t * NUM_SUBCORES) * CHUNK, 8)

    def issue_load(t, b):
        q = window_q(t)
        pltpu.async_copy(x_hbm.at[pl.ds(q, CHUNK)], xbufs[b], semxs[b])
        pltpu.async_copy(batch_hbm.at[pl.ds(q, CHUNK)], ibufs[b], semis[b])

    for b in range(NRING - 1):
        @pl.when(b < num_windows)
        def _():
            issue_load(b, b)

    def tri(g, carry):
        for b in range(NRING):
            t = NRING * g + b

            @pl.when(t < num_windows)
            def _():
                p = (b + NRING - 1) % NRING

                @pl.when(t >= 1)
                def _():
                    pltpu.make_async_copy(
                        xbufs[p], acc.at[ibufs[p]], semss[p]
                    ).wait()

                @pl.when(t + NRING - 1 < num_windows)
                def _():
                    issue_load(t + NRING - 1, p)

                q = window_q(t)
                pltpu.make_async_copy(
                    batch_hbm.at[pl.ds(q, CHUNK)], ibufs[b], semis[b]
                ).wait()
                pltpu.make_async_copy(
                    x_hbm.at[pl.ds(q, CHUNK)], xbufs[b], semxs[b]
                ).wait()
                pltpu.async_copy(
                    xbufs[b], acc.at[ibufs[b]], semss[b], add=True
                )

        return carry

    lax.fori_loop(0, (num_windows + NRING - 1) // NRING, tri, 0)

    for k in range(NRING):
        @pl.when((num_windows > 0) & ((num_windows - 1) % NRING == k))
        def _():
            pltpu.make_async_copy(xbufs[k], acc.at[ibufs[k]], semss[k]).wait()

    plsc.subcore_barrier()

    @pl.when(c == 0)
    def _():
        pltpu.sync_copy(
            acc.at[pl.ds(ZROWS * s, ZROWS)], a0_hbm.at[pl.ds(ZROWS * s, ZROWS)]
        )

    @pl.when(c == 1)
    def _():
        pltpu.sync_copy(
            acc.at[pl.ds(ZROWS * s, ZROWS)], a1_hbm.at[pl.ds(ZROWS * s, ZROWS)]
        )


def _add_body(a_ref, b_ref, o_ref):
    o_ref[...] = a_ref[...] + b_ref[...]


@jax.jit
def kernel(x, batch):
    batch = batch.astype(jnp.int32)
    zeros = jnp.zeros((CHUNK, D), jnp.float32)

    mesh = plsc.VectorSubcoreMesh(core_axis_name="c", subcore_axis_name="s")
    phase1 = pl.kernel(
        _phase1_body,
        mesh=mesh,
        out_type=(
            jax.ShapeDtypeStruct((ACC_ROWS, D), jnp.float32),
            jax.ShapeDtypeStruct((ACC_ROWS, D), jnp.float32),
        ),
        scratch_types=[
            pltpu.VMEM((CHUNK, D), jnp.float32),
            pltpu.VMEM((CHUNK, D), jnp.float32),
            pltpu.VMEM((CHUNK, D), jnp.float32),
            pltpu.VMEM((CHUNK,), jnp.int32),
            pltpu.VMEM((CHUNK,), jnp.int32),
            pltpu.VMEM((CHUNK,), jnp.int32),
            pltpu.VMEM_SHARED((ACC_ROWS, D), jnp.float32),
            pltpu.SemaphoreType.DMA,
            pltpu.SemaphoreType.DMA,
            pltpu.SemaphoreType.DMA,
            pltpu.SemaphoreType.DMA,
            pltpu.SemaphoreType.DMA,
            pltpu.SemaphoreType.DMA,
            pltpu.SemaphoreType.DMA,
            pltpu.SemaphoreType.DMA,
            pltpu.SemaphoreType.DMA,
        ],
    )
    a0, a1 = phase1(x, batch, zeros)

    blk = 2000
    out = pl.pallas_call(
        _add_body,
        grid=(NUM_SEG // blk,),
        in_specs=[
            pl.BlockSpec((blk, D), lambda i: (i, 0)),
            pl.BlockSpec((blk, D), lambda i: (i, 0)),
        ],
        out_specs=pl.BlockSpec((blk, D), lambda i: (i, 0)),
        out_shape=jax.ShapeDtypeStruct((NUM_SEG, D), jnp.float32),
    )(a0, a1)
    return out
